# baseline (speedup 1.0000x reference)
import functools

import jax
import jax.numpy as jnp
from jax import lax
from jax.experimental import pallas as pl
from jax.experimental.pallas import tpu as pltpu
from jax.experimental.pallas import tpu_sc as plsc

L = 16
NS = 16
NC = 2
RPT = 640
NPAD = NS * RPT
DH = 64
CH = 128
NCH = 164
EPAD = NS * NCH * CH
NVEC = DH // L
NM = RPT // CH


def _sc_body(K, x_hbm, src_hbm, dst_hbm, coef_hbm, out_hbm, t_hbm,
             acc_sh, deg_sh,
             src_v, dst_v, ba0, bh0, ba1, bh1, zbuf, dis_v, ones_v,
             coef_v, sg0, sg1, sr0, sr1, sw0, sw1):
    c = lax.axis_index("c")
    s = lax.axis_index("s")
    row0 = s * RPT
    feat0 = c * DH
    tbase = c * NPAD

    sets = ((ba0, bh0, sr0, sw0), (ba1, bh1, sr1, sw1))

    zeros16 = jnp.zeros((L,), jnp.float32)
    ones16 = jnp.ones((L,), jnp.float32)

    def z_body(i, carry):
        for v in range(NVEC):
            zbuf[i, pl.ds(v * L, L)] = zeros16
        return carry
    lax.fori_loop(0, CH, z_body, 0)

    for i in range(CH // L):
        ones_v[pl.ds(i * L, L)] = ones16

    def zdis_body(i, carry):
        dis_v[pl.ds(i * L, L)] = zeros16
        return carry
    lax.fori_loop(0, RPT // L + 1, zdis_body, 0)

    pltpu.sync_copy(src_hbm.at[s], src_v)
    pltpu.sync_copy(dst_hbm.at[s], dst_v)
    pltpu.sync_copy(coef_hbm, coef_v)

    toff = (tbase * jnp.ones((L,), jnp.int32)).astype(jnp.int32)

    def soff_body(j, carry):
        for i in range(CH // L):
            sl = pl.ds(i * L, L)
            src_v[j, sl] = src_v[j, sl] + toff
        return carry
    lax.fori_loop(0, NCH, soff_body, 0)

    pltpu.sync_copy(dis_v.at[pl.ds(0, RPT)], deg_sh.at[pl.ds(row0, RPT)])
    plsc.subcore_barrier()

    def deg_body(j, carry):
        pltpu.sync_copy(ones_v, deg_sh.at[dst_v.at[j]], add=True)
        return carry
    lax.fori_loop(0, NCH, deg_body, 0)
    plsc.subcore_barrier()

    pltpu.sync_copy(deg_sh.at[pl.ds(row0, RPT)], dis_v.at[pl.ds(0, RPT)])

    def rsq_body(i, carry):
        d = dis_v[pl.ds(i * L, L)]
        ib = plsc.bitcast(d, jnp.int32)
        ib = 0x5F3759DF - lax.shift_right_logical(ib, 1)
        y = plsc.bitcast(ib, jnp.float32)
        half = d * 0.5
        for _ in range(3):
            y = y * (1.5 - half * y * y)
        dis_v[pl.ds(i * L, L)] = y
        return carry
    lax.fori_loop(0, RPT // L, rsq_body, 0)

    c0 = coef_v[pl.ds(0, L)][0]
    for m in range(NM):
        r0 = m * CH
        pltpu.sync_copy(
            x_hbm.at[pl.ds(row0 + r0, CH), pl.ds(feat0, DH)], ba0)

        def init_body(r, carry):
            d = dis_v[pl.ds(r0 + r, L)][0]
            for v in range(NVEC):
                sl = pl.ds(v * L, L)
                xv = ba0[r, sl]
                bh0[r, sl] = c0 * xv
                ba0[r, sl] = d * xv
            return carry
        lax.fori_loop(0, CH, init_body, 0)
        pltpu.sync_copy(ba0, t_hbm.at[pl.ds(tbase + row0 + r0, CH)])
        pltpu.sync_copy(
            bh0, out_hbm.at[pl.ds(row0 + r0, CH), pl.ds(feat0, DH)])
        pltpu.sync_copy(zbuf, acc_sh.at[pl.ds(row0 + r0, CH)])
    plsc.subcore_barrier()

    def issue_reads(m, st):
        ba, bh, sr, so = st
        r0 = m * CH
        pltpu.async_copy(acc_sh.at[pl.ds(row0 + r0, CH)], ba, sr)
        pltpu.async_copy(
            out_hbm.at[pl.ds(row0 + r0, CH), pl.ds(feat0, DH)], bh, so)

    def wait_reads(m, st):
        ba, bh, sr, so = st
        r0 = m * CH
        pltpu.make_async_copy(
            acc_sh.at[pl.ds(row0 + r0, CH)], ba, sr).wait()
        pltpu.make_async_copy(
            out_hbm.at[pl.ds(row0 + r0, CH), pl.ds(feat0, DH)], bh,
            so).wait()

    def issue_writes(m, st):
        ba, bh, _, sw = st
        r0 = m * CH
        pltpu.sync_copy(ba, t_hbm.at[pl.ds(tbase + row0 + r0, CH)])
        pltpu.sync_copy(
            bh, out_hbm.at[pl.ds(row0 + r0, CH), pl.ds(feat0, DH)])
        pltpu.sync_copy(zbuf, acc_sh.at[pl.ds(row0 + r0, CH)])

    def wait_writes(st):
        pass

    def iter_body(k, carry):
        ck = coef_v[pl.ds(k + 1, L)][0]

        pltpu.async_copy(t_hbm.at[src_v.at[0]], ba0, sg0)

        def edge_body(i, ecarry):
            j0 = 2 * i
            pltpu.async_copy(t_hbm.at[src_v.at[j0 + 1]], bh0, sg1)
            pltpu.make_async_copy(
                t_hbm.at[pl.ds(0, CH)], ba0, sg0).wait()
            pltpu.sync_copy(ba0, acc_sh.at[dst_v.at[j0]], add=True)

            @pl.when(i < NCH // 2 - 1)
            def _prefetch():
                pltpu.async_copy(t_hbm.at[src_v.at[j0 + 2]], ba0, sg0)

            pltpu.make_async_copy(
                t_hbm.at[pl.ds(0, CH)], bh0, sg1).wait()
            pltpu.sync_copy(bh0, acc_sh.at[dst_v.at[j0 + 1]], add=True)
            return ecarry
        lax.fori_loop(0, NCH // 2, edge_body, 0)
        plsc.subcore_barrier()

        def compute(m, st):
            ba, bh, _, _ = st
            r0 = m * CH

            def node_body(r, ncarry):
                d = dis_v[pl.ds(r0 + r, L)][0]
                for v in range(NVEC):
                    sl = pl.ds(v * L, L)
                    h = d * ba[r, sl]
                    bh[r, sl] = bh[r, sl] + ck * h
                    ba[r, sl] = d * h
                return ncarry
            lax.fori_loop(0, CH, node_body, 0)

        issue_reads(0, sets[0])
        for m in range(NM):
            st = sets[m % 2]
            wait_reads(m, st)
            if m + 1 < NM:
                if m >= 1:
                    wait_writes(sets[(m + 1) % 2])
                issue_reads(m + 1, sets[(m + 1) % 2])
            compute(m, st)
            issue_writes(m, st)
        wait_writes(sets[(NM - 2) % 2])
        wait_writes(sets[(NM - 1) % 2])
        plsc.subcore_barrier()
        return carry
    lax.fori_loop(0, K, iter_body, 0)


@functools.partial(jax.jit, static_argnames=("K",))
def _run(x_pad, src_idx, dst_idx, coeffs_pad, K):
    mesh = plsc.VectorSubcoreMesh(core_axis_name="c", subcore_axis_name="s")
    f32 = jnp.float32
    fn = pl.kernel(
        functools.partial(_sc_body, K),
        out_type=jax.ShapeDtypeStruct((NPAD, NC * DH), f32),
        mesh=mesh,
        compiler_params=pltpu.CompilerParams(
            use_tc_tiling_on_sc=False, needs_layout_passes=False),
        scratch_types=[
            pltpu.HBM((NC * NPAD, DH), f32),
            pltpu.VMEM_SHARED((NPAD, DH), f32),
            pltpu.VMEM_SHARED((NPAD,), f32),
            pltpu.VMEM((NCH, CH), jnp.int32),
            pltpu.VMEM((NCH, CH), jnp.int32),
            pltpu.VMEM((CH, DH), f32),
            pltpu.VMEM((CH, DH), f32),
            pltpu.VMEM((CH, DH), f32),
            pltpu.VMEM((CH, DH), f32),
            pltpu.VMEM((CH, DH), f32),
            pltpu.VMEM((RPT + L,), f32),
            pltpu.VMEM((CH,), f32),
            pltpu.VMEM((32,), f32),
            pltpu.SemaphoreType.DMA,
            pltpu.SemaphoreType.DMA,
            pltpu.SemaphoreType.DMA,
            pltpu.SemaphoreType.DMA,
            pltpu.SemaphoreType.DMA,
            pltpu.SemaphoreType.DMA,
        ],
    )
    return fn(x_pad, src_idx, dst_idx, coeffs_pad)


def kernel(x, edge_index, coeffs):
    n, d = x.shape
    k = coeffs.shape[0] - 1
    pad_node = n

    src = edge_index[0].astype(jnp.int32)
    dst = edge_index[1].astype(jnp.int32)
    loop = jnp.arange(n, dtype=jnp.int32)
    src = jnp.concatenate([src, loop])
    dst = jnp.concatenate([dst, loop])
    pad_e = EPAD - src.shape[0]
    src = jnp.pad(src, (0, pad_e), constant_values=pad_node)
    dst = jnp.pad(dst, (0, pad_e), constant_values=pad_node)
    src_idx = src.reshape(NS, NCH, CH)
    dst_idx = dst.reshape(NS, NCH, CH)

    x_pad = jnp.pad(x, ((0, NPAD - n), (0, 0)))
    coeffs_pad = jnp.pad(coeffs.astype(jnp.float32), (0, 32 - (k + 1)))

    out = _run(x_pad, src_idx, dst_idx, coeffs_pad, k)
    return out[:n]


# speedup vs baseline: 1.5017x; 1.5017x over previous
import functools

import jax
import jax.numpy as jnp
from jax import lax
from jax.experimental import pallas as pl
from jax.experimental.pallas import tpu as pltpu
from jax.experimental.pallas import tpu_sc as plsc

L = 16
NS = 16
NC = 2
RPT = 640
NPAD = NS * RPT
DH = 64
CH = 128
NCH = 162
EPAD = NS * NCH * CH
NVEC = DH // L


def _sc_body(K, x_hbm, src_hbm, dst_hbm, coef_hbm, out_hbm, t_hbm,
             acc_sh, deg_sh,
             src_v, dst_v, buf0, buf1, bufh, zbuf, dis_v, ones_v, coef_v,
             sem_g0, sem_g1):
    c = lax.axis_index("c")
    s = lax.axis_index("s")
    row0 = s * RPT
    feat0 = c * DH
    tbase = c * NPAD

    zeros16 = jnp.zeros((L,), jnp.float32)
    ones16 = jnp.ones((L,), jnp.float32)

    def z_body(i, carry):
        for v in range(NVEC):
            zbuf[i, pl.ds(v * L, L)] = zeros16
        return carry
    lax.fori_loop(0, CH, z_body, 0)

    for i in range(CH // L):
        ones_v[pl.ds(i * L, L)] = ones16

    def zdis_body(i, carry):
        dis_v[pl.ds(i * L, L)] = zeros16
        return carry
    lax.fori_loop(0, RPT // L + 1, zdis_body, 0)

    pltpu.sync_copy(src_hbm.at[s], src_v)
    pltpu.sync_copy(dst_hbm.at[s], dst_v)
    pltpu.sync_copy(coef_hbm, coef_v)

    toff = (tbase * jnp.ones((L,), jnp.int32)).astype(jnp.int32)

    def soff_body(j, carry):
        for i in range(CH // L):
            sl = pl.ds(i * L, L)
            src_v[j, sl] = src_v[j, sl] + toff
        return carry
    lax.fori_loop(0, NCH, soff_body, 0)

    pltpu.sync_copy(dis_v.at[pl.ds(0, RPT)], deg_sh.at[pl.ds(row0, RPT)])
    plsc.subcore_barrier()

    def deg_body(j, carry):
        pltpu.sync_copy(ones_v, deg_sh.at[dst_v.at[j]], add=True)
        return carry
    lax.fori_loop(0, NCH, deg_body, 0)
    plsc.subcore_barrier()

    pltpu.sync_copy(deg_sh.at[pl.ds(row0, RPT)], dis_v.at[pl.ds(0, RPT)])

    def rsq_body(i, carry):
        d = dis_v[pl.ds(i * L, L)]
        ib = plsc.bitcast(d, jnp.int32)
        ib = 0x5F3759DF - lax.shift_right_logical(ib, 1)
        y = plsc.bitcast(ib, jnp.float32)
        half = d * 0.5
        for _ in range(3):
            y = y * (1.5 - half * y * y)
        dis_v[pl.ds(i * L, L)] = y
        return carry
    lax.fori_loop(0, RPT // L, rsq_body, 0)

    c0 = coef_v[pl.ds(0, L)][0]
    for m in range(RPT // CH):
        r0 = m * CH
        pltpu.sync_copy(
            x_hbm.at[pl.ds(row0 + r0, CH), pl.ds(feat0, DH)], buf0)

        def init_body(r, carry):
            d = dis_v[pl.ds(r0 + r, L)][0]
            for v in range(NVEC):
                sl = pl.ds(v * L, L)
                xv = buf0[r, sl]
                bufh[r, sl] = c0 * xv
                buf0[r, sl] = d * xv
            return carry
        lax.fori_loop(0, CH, init_body, 0)
        pltpu.sync_copy(buf0, t_hbm.at[pl.ds(tbase + row0 + r0, CH)])
        pltpu.sync_copy(
            bufh, out_hbm.at[pl.ds(row0 + r0, CH), pl.ds(feat0, DH)])
        pltpu.sync_copy(zbuf, acc_sh.at[pl.ds(row0 + r0, CH)])
    plsc.subcore_barrier()

    def iter_body(k, carry):
        ck = coef_v[pl.ds(k + 1, L)][0]

        pltpu.async_copy(t_hbm.at[src_v.at[0]], buf0, sem_g0)

        def edge_body(i, ecarry):
            j0 = 2 * i
            pltpu.async_copy(t_hbm.at[src_v.at[j0 + 1]], buf1, sem_g1)
            pltpu.make_async_copy(
                t_hbm.at[pl.ds(0, CH)], buf0, sem_g0).wait()
            pltpu.sync_copy(buf0, acc_sh.at[dst_v.at[j0]], add=True)

            @pl.when(i < NCH // 2 - 1)
            def _prefetch():
                pltpu.async_copy(t_hbm.at[src_v.at[j0 + 2]], buf0, sem_g0)

            pltpu.make_async_copy(
                t_hbm.at[pl.ds(0, CH)], buf1, sem_g1).wait()
            pltpu.sync_copy(buf1, acc_sh.at[dst_v.at[j0 + 1]], add=True)
            return ecarry
        lax.fori_loop(0, NCH // 2, edge_body, 0)
        plsc.subcore_barrier()

        for m in range(RPT // CH):
            r0 = m * CH
            pltpu.sync_copy(acc_sh.at[pl.ds(row0 + r0, CH)], buf1)
            pltpu.sync_copy(
                out_hbm.at[pl.ds(row0 + r0, CH), pl.ds(feat0, DH)], bufh)

            def node_body(r, ncarry):
                d = dis_v[pl.ds(r0 + r, L)][0]
                for v in range(NVEC):
                    sl = pl.ds(v * L, L)
                    h = d * buf1[r, sl]
                    bufh[r, sl] = bufh[r, sl] + ck * h
                    buf1[r, sl] = d * h
                return ncarry
            lax.fori_loop(0, CH, node_body, 0)
            pltpu.sync_copy(buf1, t_hbm.at[pl.ds(tbase + row0 + r0, CH)])
            pltpu.sync_copy(
                bufh, out_hbm.at[pl.ds(row0 + r0, CH), pl.ds(feat0, DH)])
            pltpu.sync_copy(zbuf, acc_sh.at[pl.ds(row0 + r0, CH)])
        plsc.subcore_barrier()
        return carry
    lax.fori_loop(0, K, iter_body, 0)


@functools.partial(jax.jit, static_argnames=("K",))
def _run(x_pad, src_idx, dst_idx, coeffs_pad, K):
    mesh = plsc.VectorSubcoreMesh(core_axis_name="c", subcore_axis_name="s")
    f32 = jnp.float32
    fn = pl.kernel(
        functools.partial(_sc_body, K),
        out_type=jax.ShapeDtypeStruct((NPAD, NC * DH), f32),
        mesh=mesh,
        compiler_params=pltpu.CompilerParams(
            use_tc_tiling_on_sc=False, needs_layout_passes=False),
        scratch_types=[
            pltpu.HBM((NC * NPAD, DH), f32),
            pltpu.VMEM_SHARED((NPAD, DH), f32),
            pltpu.VMEM_SHARED((NPAD,), f32),
            pltpu.VMEM((NCH, CH), jnp.int32),
            pltpu.VMEM((NCH, CH), jnp.int32),
            pltpu.VMEM((CH, DH), f32),
            pltpu.VMEM((CH, DH), f32),
            pltpu.VMEM((CH, DH), f32),
            pltpu.VMEM((CH, DH), f32),
            pltpu.VMEM((RPT + L,), f32),
            pltpu.VMEM((CH,), f32),
            pltpu.VMEM((32,), f32),
            pltpu.SemaphoreType.DMA,
            pltpu.SemaphoreType.DMA,
        ],
    )
    return fn(x_pad, src_idx, dst_idx, coeffs_pad)


def kernel(x, edge_index, coeffs):
    n, d = x.shape
    k = coeffs.shape[0] - 1
    pad_node = n

    src = edge_index[0].astype(jnp.int32)
    dst = edge_index[1].astype(jnp.int32)
    loop = jnp.arange(n, dtype=jnp.int32)
    src = jnp.concatenate([src, loop])
    dst = jnp.concatenate([dst, loop])
    pad_e = EPAD - src.shape[0]
    src = jnp.pad(src, (0, pad_e), constant_values=pad_node)
    dst = jnp.pad(dst, (0, pad_e), constant_values=pad_node)
    src_idx = src.reshape(NS, NCH, CH)
    dst_idx = dst.reshape(NS, NCH, CH)

    x_pad = jnp.pad(x, ((0, NPAD - n), (0, 0)))
    coeffs_pad = jnp.pad(coeffs.astype(jnp.float32), (0, 32 - (k + 1)))

    out = _run(x_pad, src_idx, dst_idx, coeffs_pad, k)
    return out[:n]


# speedup vs baseline: 1.6694x; 1.1117x over previous
import functools

import jax
import jax.numpy as jnp
from jax import lax
from jax.experimental import pallas as pl
from jax.experimental.pallas import tpu as pltpu
from jax.experimental.pallas import tpu_sc as plsc

L = 16
NS = 16
NC = 2
RPT = 640
NPAD = NS * RPT
DH = 64
CH = 128
NCH = 162
EPAD = NS * NCH * CH
NVEC = DH // L


def _sc_body(K, x_hbm, pk_hbm, coef_hbm, out_hbm,
             t_sh, acc_sh, deg_sh,
             pk_v, buf0, buf1, zbuf, dis_v, ones_v, coef_v,
             si0, si1, di0, sem_g0, sem_g1):
    bufh = buf0
    c = lax.axis_index("c")
    s = lax.axis_index("s")
    row0 = s * RPT
    feat0 = c * DH

    zeros16 = jnp.zeros((L,), jnp.float32)
    ones16 = jnp.ones((L,), jnp.float32)

    def z_body(i, carry):
        for v in range(NVEC):
            zbuf[i, pl.ds(v * L, L)] = zeros16
        return carry
    lax.fori_loop(0, CH, z_body, 0)

    for i in range(CH // L):
        ones_v[pl.ds(i * L, L)] = ones16

    def zdis_body(i, carry):
        dis_v[pl.ds(i * L, L)] = zeros16
        return carry
    lax.fori_loop(0, RPT // L + 1, zdis_body, 0)

    pltpu.sync_copy(pk_hbm.at[s], pk_v)
    pltpu.sync_copy(coef_hbm, coef_v)

    mask14 = jnp.full((L,), 0x3FFF, jnp.int32)

    def unpack_src(j, ib):
        for i in range(CH // L):
            sl = pl.ds(i * L, L)
            ib[sl] = pk_v[j, sl] & mask14

    def unpack_dst(j, ib):
        for i in range(CH // L):
            sl = pl.ds(i * L, L)
            ib[sl] = lax.shift_right_logical(pk_v[j, sl], 14)

    pltpu.sync_copy(dis_v.at[pl.ds(0, RPT)], deg_sh.at[pl.ds(row0, RPT)])
    plsc.subcore_barrier()

    def deg_body(j, carry):
        unpack_dst(j, di0)
        pltpu.sync_copy(ones_v, deg_sh.at[di0], add=True)
        return carry
    lax.fori_loop(0, NCH, deg_body, 0)
    plsc.subcore_barrier()

    pltpu.sync_copy(deg_sh.at[pl.ds(row0, RPT)], dis_v.at[pl.ds(0, RPT)])

    def rsq_body(i, carry):
        d = dis_v[pl.ds(i * L, L)]
        ib = plsc.bitcast(d, jnp.int32)
        ib = 0x5F3759DF - lax.shift_right_logical(ib, 1)
        y = plsc.bitcast(ib, jnp.float32)
        half = d * 0.5
        for _ in range(3):
            y = y * (1.5 - half * y * y)
        dis_v[pl.ds(i * L, L)] = y
        return carry
    lax.fori_loop(0, RPT // L, rsq_body, 0)

    c0 = coef_v[pl.ds(0, L)][0]
    for m in range(RPT // CH):
        r0 = m * CH
        pltpu.sync_copy(
            x_hbm.at[pl.ds(row0 + r0, CH), pl.ds(feat0, DH)], buf1)

        def init_body(r, carry):
            d = dis_v[pl.ds(r0 + r, L)][0]
            for v in range(NVEC):
                sl = pl.ds(v * L, L)
                xv = buf1[r, sl]
                bufh[r, sl] = c0 * xv
                buf1[r, sl] = d * xv
            return carry
        lax.fori_loop(0, CH, init_body, 0)
        pltpu.sync_copy(buf1, t_sh.at[pl.ds(row0 + r0, CH)])
        pltpu.sync_copy(
            bufh, out_hbm.at[pl.ds(row0 + r0, CH), pl.ds(feat0, DH)])
        pltpu.sync_copy(zbuf, acc_sh.at[pl.ds(row0 + r0, CH)])
    plsc.subcore_barrier()

    def iter_body(k, carry):
        ck = coef_v[pl.ds(k + 1, L)][0]

        unpack_src(0, si0)
        pltpu.async_copy(t_sh.at[si0], buf0, sem_g0)

        def edge_body(i, ecarry):
            j0 = 2 * i
            unpack_src(j0 + 1, si1)
            pltpu.async_copy(t_sh.at[si1], buf1, sem_g1)
            pltpu.make_async_copy(
                t_sh.at[pl.ds(0, CH)], buf0, sem_g0).wait()
            unpack_dst(j0, di0)
            pltpu.sync_copy(buf0, acc_sh.at[di0], add=True)

            @pl.when(i < NCH // 2 - 1)
            def _prefetch():
                unpack_src(j0 + 2, si0)
                pltpu.async_copy(t_sh.at[si0], buf0, sem_g0)

            pltpu.make_async_copy(
                t_sh.at[pl.ds(0, CH)], buf1, sem_g1).wait()
            unpack_dst(j0 + 1, di0)
            pltpu.sync_copy(buf1, acc_sh.at[di0], add=True)
            return ecarry
        lax.fori_loop(0, NCH // 2, edge_body, 0)
        plsc.subcore_barrier()

        for m in range(RPT // CH):
            r0 = m * CH
            pltpu.sync_copy(acc_sh.at[pl.ds(row0 + r0, CH)], buf1)
            pltpu.sync_copy(
                out_hbm.at[pl.ds(row0 + r0, CH), pl.ds(feat0, DH)], bufh)

            def node_body(r, ncarry):
                d = dis_v[pl.ds(r0 + r, L)][0]
                for v in range(NVEC):
                    sl = pl.ds(v * L, L)
                    h = d * buf1[r, sl]
                    bufh[r, sl] = bufh[r, sl] + ck * h
                    buf1[r, sl] = d * h
                return ncarry
            lax.fori_loop(0, CH, node_body, 0)
            pltpu.sync_copy(buf1, t_sh.at[pl.ds(row0 + r0, CH)])
            pltpu.sync_copy(
                bufh, out_hbm.at[pl.ds(row0 + r0, CH), pl.ds(feat0, DH)])
            pltpu.sync_copy(zbuf, acc_sh.at[pl.ds(row0 + r0, CH)])
        plsc.subcore_barrier()
        return carry
    lax.fori_loop(0, K, iter_body, 0)


@functools.partial(jax.jit, static_argnames=("K",))
def _run(x_pad, packed_idx, coeffs_pad, K):
    mesh = plsc.VectorSubcoreMesh(core_axis_name="c", subcore_axis_name="s")
    f32 = jnp.float32
    fn = pl.kernel(
        functools.partial(_sc_body, K),
        out_type=jax.ShapeDtypeStruct((NPAD, NC * DH), f32),
        mesh=mesh,
        compiler_params=pltpu.CompilerParams(
            use_tc_tiling_on_sc=False, needs_layout_passes=False),
        scratch_types=[
            pltpu.VMEM_SHARED((NPAD, DH), f32),
            pltpu.VMEM_SHARED((NPAD, DH), f32),
            pltpu.VMEM_SHARED((NPAD,), f32),
            pltpu.VMEM((NCH, CH), jnp.int32),
            pltpu.VMEM((CH, DH), f32),
            pltpu.VMEM((CH, DH), f32),
            pltpu.VMEM((CH, DH), f32),
            pltpu.VMEM((RPT + L,), f32),
            pltpu.VMEM((CH,), f32),
            pltpu.VMEM((32,), f32),
            pltpu.VMEM((CH,), jnp.int32),
            pltpu.VMEM((CH,), jnp.int32),
            pltpu.VMEM((CH,), jnp.int32),
            pltpu.SemaphoreType.DMA,
            pltpu.SemaphoreType.DMA,
        ],
    )
    return fn(x_pad, packed_idx, coeffs_pad)


def kernel(x, edge_index, coeffs):
    n, d = x.shape
    k = coeffs.shape[0] - 1
    pad_node = n

    src = edge_index[0].astype(jnp.int32)
    dst = edge_index[1].astype(jnp.int32)
    loop = jnp.arange(n, dtype=jnp.int32)
    src = jnp.concatenate([src, loop])
    dst = jnp.concatenate([dst, loop])
    pad_e = EPAD - src.shape[0]
    src = jnp.pad(src, (0, pad_e), constant_values=pad_node)
    dst = jnp.pad(dst, (0, pad_e), constant_values=pad_node)
    packed = (src | (dst << 14)).reshape(NS, NCH, CH)

    x_pad = jnp.pad(x, ((0, NPAD - n), (0, 0)))
    coeffs_pad = jnp.pad(coeffs.astype(jnp.float32), (0, 32 - (k + 1)))

    out = _run(x_pad, packed, coeffs_pad, k)
    return out[:n]
